# baseline (speedup 1.0000x reference)
import functools

import jax
import jax.numpy as jnp
from jax import lax
from jax.experimental import pallas as pl
from jax.experimental.pallas import tpu as pltpu
from jax.experimental.pallas import tpu_sc as plsc

ROWS = 128
COLS = 32768
NC = 2
NS = 16
L = 16
NW = NC * NS
RPW = ROWS // NW
VPR = COLS // L
UNROLL = 16
ITERS = VPR // UNROLL
FMIN = -3.402823466e38
FMAX = 3.402823466e38

_mesh = plsc.VectorSubcoreMesh(core_axis_name="c", subcore_axis_name="s")


@functools.partial(
    pl.kernel,
    mesh=_mesh,
    out_type=jax.ShapeDtypeStruct((NW, L), jnp.float32),
    scratch_types=[
        pltpu.VMEM((2, COLS), jnp.float32),
        pltpu.VMEM((ROWS + L,), jnp.float32),
        pltpu.VMEM((L,), jnp.float32),
        pltpu.SemaphoreType.DMA,
        pltpu.SemaphoreType.DMA,
    ],
)
def _absmax_gate(neuron_hbm, seg_hbm, out_hbm, buf, neu_v, res_v, sem0, sem1):
    wid = lax.axis_index("s") * NC + lax.axis_index("c")
    row0 = wid * RPW
    pltpu.sync_copy(neuron_hbm, neu_v.at[pl.ds(0, ROWS)])
    neu_v[pl.ds(ROWS, L)] = jnp.zeros((L,), jnp.float32)

    sems = (sem0, sem1)
    copies = [pltpu.async_copy(seg_hbm.at[row0], buf.at[0], sem0), None]

    lane = jnp.arange(L, dtype=jnp.int32)
    res = jnp.zeros((L,), jnp.float32)

    for r in range(RPW):
        if r + 1 < RPW:
            nxt = (r + 1) % 2
            copies[nxt] = pltpu.async_copy(
                seg_hbm.at[row0 + r + 1], buf.at[nxt], sems[nxt])
        copies[r % 2].wait()
        b = buf.at[r % 2]

        def step(i, carry, b=b):
            m0, m1, m2, m3, n0, n1, n2, n3 = carry
            base = i * (UNROLL * L)
            v = [b[pl.ds(base + k * L, L)] for k in range(UNROLL)]
            t0 = jnp.maximum(jnp.maximum(v[0], v[1]), jnp.maximum(v[2], v[3]))
            t1 = jnp.maximum(jnp.maximum(v[4], v[5]), jnp.maximum(v[6], v[7]))
            t2 = jnp.maximum(jnp.maximum(v[8], v[9]), jnp.maximum(v[10], v[11]))
            t3 = jnp.maximum(jnp.maximum(v[12], v[13]), jnp.maximum(v[14], v[15]))
            u0 = jnp.minimum(jnp.minimum(v[0], v[1]), jnp.minimum(v[2], v[3]))
            u1 = jnp.minimum(jnp.minimum(v[4], v[5]), jnp.minimum(v[6], v[7]))
            u2 = jnp.minimum(jnp.minimum(v[8], v[9]), jnp.minimum(v[10], v[11]))
            u3 = jnp.minimum(jnp.minimum(v[12], v[13]), jnp.minimum(v[14], v[15]))
            return (jnp.maximum(m0, t0), jnp.maximum(m1, t1),
                    jnp.maximum(m2, t2), jnp.maximum(m3, t3),
                    jnp.minimum(n0, u0), jnp.minimum(n1, u1),
                    jnp.minimum(n2, u2), jnp.minimum(n3, u3))

        init = ((jnp.full((L,), FMIN, jnp.float32),) * 4
                + (jnp.full((L,), FMAX, jnp.float32),) * 4)
        m0, m1, m2, m3, n0, n1, n2, n3 = lax.fori_loop(0, ITERS, step, init)
        m = jnp.maximum(jnp.maximum(m0, m1), jnp.maximum(m2, m3))
        n = jnp.minimum(jnp.minimum(n0, n1), jnp.minimum(n2, n3))
        for k in (8, 4, 2, 1):
            perm = jnp.bitwise_xor(lane, k)
            m = jnp.maximum(m, m.at[perm].get(mode="promise_in_bounds"))
            n = jnp.minimum(n, n.at[perm].get(mode="promise_in_bounds"))
        signed = jnp.where(m >= -n, m, n)
        res = jnp.where(lane == r, signed, res)

    neu = neu_v[pl.ds(row0, L)]
    gate = 1.0 / (1.0 + jnp.exp(-res))
    res_v[...] = neu * gate
    pltpu.sync_copy(res_v, out_hbm.at[wid])


def kernel(neuron_out, segment_out):
    padded = _absmax_gate(neuron_out, segment_out)
    return padded[:, :RPW].reshape(ROWS)
